# baseline (speedup 1.0000x reference)
import functools

import jax
import jax.numpy as jnp
from jax import lax
from jax.experimental import pallas as pl
from jax.experimental.pallas import tpu as pltpu
from jax.experimental.pallas import tpu_sc as plsc

_D = 64
_B = 4096
_S = 200
_NW = 32
_RPW = _B // _NW
_NCH = _RPW * 2
_NBUF = 8
_PRE = 4
_SPLIT = ((0, 128), (128, 72))


def _emb_body(idx_l_hbm, idx_r_hbm, table_hbm, out_hbm,
              idx_vl, idx_vr, rows_v, gsem, wsem):
    nc = plsc.get_sparse_core_info().num_cores
    wid = lax.axis_index("s") * nc + lax.axis_index("c")
    r0 = wid * _RPW

    pltpu.sync_copy(idx_l_hbm.at[pl.ds(r0, _RPW)], idx_vl)
    pltpu.sync_copy(idx_r_hbm.at[pl.ds(r0, _RPW)], idx_vr)

    def gather(c, b):
        n = _SPLIT[b % 2][1]
        src = idx_vl.at[c // 2] if b % 2 == 0 else idx_vr.at[c // 2, pl.ds(0, n)]
        pltpu.async_copy(table_hbm.at[src], rows_v.at[b, pl.ds(0, n)], gsem.at[b])

    def write(c, b):
        s0, n = _SPLIT[b % 2]
        pltpu.async_copy(
            rows_v.at[b, pl.ds(0, n)],
            out_hbm.at[r0 + c // 2, pl.ds(s0, n)],
            wsem.at[b],
        )

    def wait_g(b):
        n = _SPLIT[b % 2][1]
        pltpu.make_async_copy(
            table_hbm.at[pl.ds(0, n)], rows_v.at[b, pl.ds(0, n)], gsem.at[b]
        ).wait()

    def wait_w(b):
        n = _SPLIT[b % 2][1]
        pltpu.make_async_copy(
            rows_v.at[b, pl.ds(0, n)],
            out_hbm.at[0, pl.ds(0, n)],
            wsem.at[b],
        ).wait()

    for g in range(_PRE):
        gather(g, g)

    for c in range(_NBUF - _PRE):
        wait_g(c)
        write(c, c)
        gather(c + _PRE, c + _PRE)

    def outer(i, carry):
        base = (_NBUF - _PRE) + i * _NBUF
        for j in range(_NBUF):
            c = base + j
            b = (_NBUF - _PRE + j) % _NBUF
            bg = j
            wait_g(b)
            write(c, b)
            wait_w(bg)
            gather(c + _PRE, bg)
        return carry

    lax.fori_loop(0, (_NCH - _NBUF) // _NBUF, outer, 0)

    for j in range(_PRE):
        c = _NCH - _PRE + j
        wait_g(c % _NBUF)
        write(c, c % _NBUF)

    for j in range(_NBUF):
        wait_w((_NCH - _NBUF + j) % _NBUF)


_emb = functools.partial(
    pl.kernel,
    mesh=plsc.VectorSubcoreMesh(core_axis_name="c", subcore_axis_name="s"),
    out_type=jax.ShapeDtypeStruct((_B, _S, _D), jnp.float32),
    scratch_types=[
        pltpu.VMEM((_RPW, 128), jnp.int32),
        pltpu.VMEM((_RPW, 128), jnp.int32),
        pltpu.VMEM((_NBUF, 128, _D), jnp.float32),
        pltpu.SemaphoreType.DMA((_NBUF,)),
        pltpu.SemaphoreType.DMA((_NBUF,)),
    ],
    compiler_params=pltpu.CompilerParams(use_tc_tiling_on_sc=False),
)(_emb_body)


def kernel(token_ids, embedding_matrix):
    idx = token_ids.astype(jnp.int32)
    idx_l = idx[:, :128]
    idx_r = jnp.pad(idx[:, 128:], ((0, 0), (0, 56)))
    return _emb(idx_l, idx_r, embedding_matrix)


# speedup vs baseline: 1.0013x; 1.0013x over previous
import functools

import jax
import jax.numpy as jnp
from jax import lax
from jax.experimental import pallas as pl
from jax.experimental.pallas import tpu as pltpu
from jax.experimental.pallas import tpu_sc as plsc

_D = 64
_B = 4096
_S = 200
_SP = 256
_NW = 32
_RPW = _B // _NW
_NCH = _RPW * 2
_NBUF = 8
_PRE = 4
_SPLIT = ((0, 128), (128, 72))


def _emb_body(idx_hbm, table_hbm, out_hbm, idx_v, rows_v, gsem, wsem):
    nc = plsc.get_sparse_core_info().num_cores
    wid = lax.axis_index("s") * nc + lax.axis_index("c")
    r0 = wid * _RPW

    pltpu.sync_copy(idx_hbm.at[pl.ds(r0, _RPW)], idx_v)

    def gather(c, b):
        s0, n = _SPLIT[b % 2]
        pltpu.async_copy(
            table_hbm.at[idx_v.at[c // 2, pl.ds(s0, n)]],
            rows_v.at[b, pl.ds(0, n)],
            gsem.at[b],
        )

    def write(c, b):
        s0, n = _SPLIT[b % 2]
        pltpu.async_copy(
            rows_v.at[b, pl.ds(0, n)],
            out_hbm.at[r0 + c // 2, pl.ds(s0, n)],
            wsem.at[b],
        )

    def wait_g(b):
        n = _SPLIT[b % 2][1]
        pltpu.make_async_copy(
            table_hbm.at[pl.ds(0, n)], rows_v.at[b, pl.ds(0, n)], gsem.at[b]
        ).wait()

    def wait_w(b):
        n = _SPLIT[b % 2][1]
        pltpu.make_async_copy(
            rows_v.at[b, pl.ds(0, n)],
            out_hbm.at[0, pl.ds(0, n)],
            wsem.at[b],
        ).wait()

    for g in range(_PRE):
        gather(g, g)

    for c in range(_NBUF - _PRE):
        wait_g(c)
        write(c, c)
        gather(c + _PRE, c + _PRE)

    def outer(i, carry):
        base = (_NBUF - _PRE) + i * _NBUF
        for j in range(_NBUF):
            c = base + j
            b = (_NBUF - _PRE + j) % _NBUF
            bg = j
            wait_g(b)
            write(c, b)
            wait_w(bg)
            gather(c + _PRE, bg)
        return carry

    lax.fori_loop(0, (_NCH - _NBUF) // _NBUF, outer, 0)

    for j in range(_PRE):
        c = _NCH - _PRE + j
        wait_g(c % _NBUF)
        write(c, c % _NBUF)

    for j in range(_NBUF):
        wait_w((_NCH - _NBUF + j) % _NBUF)


_emb = functools.partial(
    pl.kernel,
    mesh=plsc.VectorSubcoreMesh(core_axis_name="c", subcore_axis_name="s"),
    out_type=jax.ShapeDtypeStruct((_B, _S, _D), jnp.float32),
    scratch_types=[
        pltpu.VMEM((_RPW, _SP), jnp.int32),
        pltpu.VMEM((_NBUF, 128, _D), jnp.float32),
        pltpu.SemaphoreType.DMA((_NBUF,)),
        pltpu.SemaphoreType.DMA((_NBUF,)),
    ],
    compiler_params=pltpu.CompilerParams(use_tc_tiling_on_sc=False),
)(_emb_body)


def kernel(token_ids, embedding_matrix):
    idx = jnp.pad(token_ids.astype(jnp.int32), ((0, 0), (0, _SP - _S)))
    return _emb(idx, embedding_matrix)


# speedup vs baseline: 1.3320x; 1.3302x over previous
import functools

import jax
import jax.numpy as jnp
from jax import lax
from jax.experimental import pallas as pl
from jax.experimental.pallas import tpu as pltpu
from jax.experimental.pallas import tpu_sc as plsc

_D = 64
_B = 4096
_S = 200
_NW = 32
_RPW = _B // _NW
_NCH = _RPW * 2
_NBUF = 8
_PRE = 4
_SPLIT = ((0, 128), (128, 72))


def _emb_body(idx_hbm, table_hbm, out_hbm, idx_v, rows_v, gsem, wsem):
    nc = plsc.get_sparse_core_info().num_cores
    wid = lax.axis_index("s") * nc + lax.axis_index("c")
    r0 = wid * _RPW

    pltpu.sync_copy(idx_hbm.at[pl.ds(r0, _RPW)], idx_v)

    def gather(c, b):
        s0, n = _SPLIT[b % 2]
        pltpu.async_copy(
            table_hbm.at[idx_v.at[c // 2, pl.ds(s0, n)]],
            rows_v.at[b, pl.ds(0, n)],
            gsem.at[b],
        )

    def write(c, b):
        s0, n = _SPLIT[b % 2]
        pltpu.async_copy(
            rows_v.at[b, pl.ds(0, n)],
            out_hbm.at[r0 + c // 2, pl.ds(s0, n), pl.ds(0, _D)],
            wsem.at[b],
        )

    def wait_g(b):
        n = _SPLIT[b % 2][1]
        pltpu.make_async_copy(
            table_hbm.at[pl.ds(0, n)], rows_v.at[b, pl.ds(0, n)], gsem.at[b]
        ).wait()

    def wait_w(b):
        n = _SPLIT[b % 2][1]
        pltpu.make_async_copy(
            rows_v.at[b, pl.ds(0, n)],
            out_hbm.at[0, pl.ds(0, n), pl.ds(0, _D)],
            wsem.at[b],
        ).wait()

    for g in range(_PRE):
        gather(g, g)

    for c in range(_NBUF - _PRE):
        wait_g(c)
        write(c, c)
        gather(c + _PRE, c + _PRE)

    def outer(i, carry):
        base = (_NBUF - _PRE) + i * _NBUF
        for j in range(_NBUF):
            c = base + j
            b = (_NBUF - _PRE + j) % _NBUF
            bg = j
            wait_g(b)
            write(c, b)
            wait_w(bg)
            gather(c + _PRE, bg)
        return carry

    lax.fori_loop(0, (_NCH - _NBUF) // _NBUF, outer, 0)

    for j in range(_PRE):
        c = _NCH - _PRE + j
        wait_g(c % _NBUF)
        write(c, c % _NBUF)

    for j in range(_NBUF):
        wait_w((_NCH - _NBUF + j) % _NBUF)


_emb = functools.partial(
    pl.kernel,
    mesh=plsc.VectorSubcoreMesh(core_axis_name="c", subcore_axis_name="s"),
    out_type=jax.ShapeDtypeStruct((_B, _S, 2 * _D), jnp.float32),
    scratch_types=[
        pltpu.VMEM((_RPW, _S), jnp.int32),
        pltpu.VMEM((_NBUF, 128, _D), jnp.float32),
        pltpu.SemaphoreType.DMA((_NBUF,)),
        pltpu.SemaphoreType.DMA((_NBUF,)),
    ],
    compiler_params=pltpu.CompilerParams(use_tc_tiling_on_sc=False),
)(_emb_body)


def kernel(token_ids, embedding_matrix):
    out = _emb(token_ids.astype(jnp.int32), embedding_matrix)
    return out[:, :, :_D]
